# baseline (speedup 1.0000x reference)
import functools
import jax
import jax.numpy as jnp
from jax.experimental import pallas as pl


N_SAMPLES = 1024
N_CLUSTERS = 1024
D_EMB = 32
LANES = 128
GROUPS = LANES // D_EMB
M_ROWS = N_CLUSTERS // GROUPS
BLK_N = 8


def _proj_kernel(ctx_ref, q_ref, w1_ref, w2_ref, b_ref, d_ref):
    proj = (
        jnp.dot(ctx_ref[...], w1_ref[...], preferred_element_type=jnp.float32)
        + jnp.dot(q_ref[...], w2_ref[...], preferred_element_type=jnp.float32)
        + b_ref[...]
    )
    d_ref[...] = jnp.concatenate([proj, proj, proj, proj], axis=1)


def _logits_kernel(cc_ref, d_ref, out_ref):
    c_iota = jax.lax.broadcasted_iota(jnp.int32, (LANES, GROUPS), 0)
    g_iota = jax.lax.broadcasted_iota(jnp.int32, (LANES, GROUPS), 1)
    G = jnp.where(c_iota // D_EMB == g_iota, 1.0, 0.0).astype(jnp.float32)
    for s in range(BLK_N):
        z = cc_ref[s] * d_ref[s : s + 1, :]
        out_ref[s] = jnp.dot(z, G, preferred_element_type=jnp.float32)


def _logsoftmax_kernel(x_ref, o_ref):
    x = x_ref[...]
    m = jnp.max(x, axis=1, keepdims=True)
    e = jnp.exp(x - m)
    s = jnp.sum(e, axis=1, keepdims=True)
    o_ref[...] = (x - m) - jnp.log(s)


@jax.jit
def kernel(context, query, cluster_centers, W, b):
    n, dc = context.shape
    w1 = W[:dc]
    w2 = W[dc:]
    b_row = b.reshape(1, -1)

    d_tiled = pl.pallas_call(
        _proj_kernel,
        out_shape=jax.ShapeDtypeStruct((n, LANES), jnp.float32),
    )(context, query, w1, w2, b_row)

    cc_r = cluster_centers.reshape(n, M_ROWS, LANES)
    grid = n // BLK_N
    logits4 = pl.pallas_call(
        _logits_kernel,
        grid=(grid,),
        in_specs=[
            pl.BlockSpec((BLK_N, M_ROWS, LANES), lambda i: (i, 0, 0)),
            pl.BlockSpec((BLK_N, LANES), lambda i: (i, 0)),
        ],
        out_specs=pl.BlockSpec((BLK_N, M_ROWS, GROUPS), lambda i: (i, 0, 0)),
        out_shape=jax.ShapeDtypeStruct((n, M_ROWS, GROUPS), jnp.float32),
    )(cc_r, d_tiled)
    logits = logits4.reshape(n, N_CLUSTERS)

    blk_r = 128
    log_probs = pl.pallas_call(
        _logsoftmax_kernel,
        grid=(n // blk_r,),
        in_specs=[pl.BlockSpec((blk_r, N_CLUSTERS), lambda i: (i, 0))],
        out_specs=pl.BlockSpec((blk_r, N_CLUSTERS), lambda i: (i, 0)),
        out_shape=jax.ShapeDtypeStruct((n, N_CLUSTERS), jnp.float32),
    )(logits)

    return (logits, log_probs)


# speedup vs baseline: 1.0133x; 1.0133x over previous
import functools
import jax
import jax.numpy as jnp
from jax.experimental import pallas as pl


N_SAMPLES = 1024
N_CLUSTERS = 1024
D_EMB = 32
LANES = 128
GROUPS = LANES // D_EMB
M_ROWS = N_CLUSTERS // GROUPS
BLK_N = 8


def _proj_kernel(ctx_ref, q_ref, w1_ref, w2_ref, b_ref, d_ref):
    proj = (
        jnp.dot(ctx_ref[...], w1_ref[...], preferred_element_type=jnp.float32)
        + jnp.dot(q_ref[...], w2_ref[...], preferred_element_type=jnp.float32)
        + b_ref[...]
    )
    d_ref[...] = jnp.concatenate([proj, proj, proj, proj], axis=1)


def _logits_kernel(cc_ref, p_ref, out_ref):
    out_ref[...] = jax.lax.dot_general(
        p_ref[...],
        cc_ref[...],
        dimension_numbers=(((1,), (2,)), ((0,), (0,))),
        preferred_element_type=jnp.float32,
    )


def _logsoftmax_kernel(x_ref, o_ref):
    x = x_ref[...]
    m = jnp.max(x, axis=1, keepdims=True)
    e = jnp.exp(x - m)
    s = jnp.sum(e, axis=1, keepdims=True)
    o_ref[...] = (x - m) - jnp.log(s)


@jax.jit
def kernel(context, query, cluster_centers, W, b):
    n, dc = context.shape
    w1 = W[:dc]
    w2 = W[dc:]
    b_row = b.reshape(1, -1)

    d_tiled = pl.pallas_call(
        _proj_kernel,
        out_shape=jax.ShapeDtypeStruct((n, LANES), jnp.float32),
    )(context, query, w1, w2, b_row)
    proj = d_tiled[:, :D_EMB]

    grid = n // BLK_N
    logits = pl.pallas_call(
        _logits_kernel,
        grid=(grid,),
        in_specs=[
            pl.BlockSpec((BLK_N, N_CLUSTERS, D_EMB), lambda i: (i, 0, 0)),
            pl.BlockSpec((BLK_N, D_EMB), lambda i: (i, 0)),
        ],
        out_specs=pl.BlockSpec((BLK_N, N_CLUSTERS), lambda i: (i, 0)),
        out_shape=jax.ShapeDtypeStruct((n, N_CLUSTERS), jnp.float32),
    )(cluster_centers, proj)

    blk_r = 128
    log_probs = pl.pallas_call(
        _logsoftmax_kernel,
        grid=(n // blk_r,),
        in_specs=[pl.BlockSpec((blk_r, N_CLUSTERS), lambda i: (i, 0))],
        out_specs=pl.BlockSpec((blk_r, N_CLUSTERS), lambda i: (i, 0)),
        out_shape=jax.ShapeDtypeStruct((n, N_CLUSTERS), jnp.float32),
    )(logits)

    return (logits, log_probs)
